# Initial kernel scaffold; baseline (speedup 1.0000x reference)
import jax
import jax.numpy as jnp
from jax.experimental import pallas as pl


def kernel(rule_features, selected_mask, expert_bias, selected_idx):
    raise NotImplementedError("write your pallas kernel here")


# speedup vs baseline: 2.2766x; 2.2766x over previous
import functools

import jax
import jax.numpy as jnp
from jax.experimental import pallas as pl

_N_TOK = 16384
_N_FEAT = 64
_N_EXPERTS = 16
_N_SEL = 4
_N_BINS = 5
_TEMPERATURE = 1.0

_BLK = 1024
_NBLK = _N_TOK // _BLK

_THR_ERF = (-0.8416212797164917, -0.25334709882736206,
            0.25334709882736206, 0.8416213393211365)
_THR_LIN = (0.20000000298023224, 0.4000000059604645,
            0.6000000238418579, 0.800000011920929)

_HI = jax.lax.Precision.HIGHEST


def _router_block(thr, emit_minmax, x_ref, onehot_ref,
                  sel0_ref, sel1_ref, sel2_ref, sel3_ref,
                  mask_ref, cnt_ref, bias_ref, *out_refs):
    if emit_minmax:
        w_ref, l_ref, lo_ref, hi_ref = out_refs
    else:
        w_ref, l_ref = out_refs

    x = x_ref[...]
    g = jnp.dot(x, onehot_ref[...], preferred_element_type=jnp.float32,
                precision=_HI)

    if emit_minmax:
        lo_ref[...] = jnp.full((1, 1, 128), jnp.min(g), jnp.float32)
        hi_ref[...] = jnp.full((1, 1, 128), jnp.max(g), jnp.float32)

    bins = (jnp.where(g >= thr[0], 1.0, 0.0) + jnp.where(g >= thr[1], 1.0, 0.0)
            + jnp.where(g >= thr[2], 1.0, 0.0) + jnp.where(g >= thr[3], 1.0, 0.0))
    bc = (bins + 0.5) / float(_N_BINS)

    p0 = jnp.dot(bc, sel0_ref[...], preferred_element_type=jnp.float32, precision=_HI)
    p1 = jnp.dot(bc, sel1_ref[...], preferred_element_type=jnp.float32, precision=_HI)
    p2 = jnp.dot(bc, sel2_ref[...], preferred_element_type=jnp.float32, precision=_HI)
    p3 = jnp.dot(bc, sel3_ref[...], preferred_element_type=jnp.float32, precision=_HI)
    m = mask_ref[...]
    t0 = p0 * m[0:1, :]
    t1 = p1 * m[1:2, :]
    t2 = p2 * m[2:3, :]
    t3 = p3 * m[3:4, :]
    acc = (t0 + t2) + (t1 + t3)
    logits = acc / cnt_ref[...] + bias_ref[...]

    scale = max(float(_TEMPERATURE), 1e-06)
    scaled = logits / scale

    cols = jax.lax.broadcasted_iota(jnp.int32, scaled.shape, 1)
    m1 = jnp.max(scaled, axis=1, keepdims=True)
    i1 = jnp.min(jnp.where(scaled == m1, cols, _N_EXPERTS), axis=1, keepdims=True)
    masked = jnp.where(cols == i1, -jnp.inf, scaled)
    m2 = jnp.max(masked, axis=1, keepdims=True)
    i2 = jnp.min(jnp.where(masked == m2, cols, _N_EXPERTS), axis=1, keepdims=True)

    b = jnp.exp(m2 - m1)
    w1 = 1.0 / (1.0 + b)
    w2 = b / (1.0 + b)
    w = jnp.where(cols == i1, w1, 0.0) + jnp.where(cols == i2, w2, 0.0)

    w_ref[...] = w
    l_ref[...] = scaled


def _small(shape):
    return pl.BlockSpec(shape, lambda i: tuple(0 for _ in shape))


def _make_call(thr, emit_minmax):
    out_shape = [
        jax.ShapeDtypeStruct((_N_TOK, _N_EXPERTS), jnp.float32),
        jax.ShapeDtypeStruct((_N_TOK, _N_EXPERTS), jnp.float32),
    ]
    out_specs = [
        pl.BlockSpec((_BLK, _N_EXPERTS), lambda i: (i, 0)),
        pl.BlockSpec((_BLK, _N_EXPERTS), lambda i: (i, 0)),
    ]
    if emit_minmax:
        out_shape += [jax.ShapeDtypeStruct((_NBLK, 1, 128), jnp.float32)] * 2
        out_specs += [pl.BlockSpec((1, 1, 128), lambda i: (i, 0, 0))] * 2
    return pl.pallas_call(
        functools.partial(_router_block, thr, emit_minmax),
        grid=(_NBLK,),
        in_specs=[
            pl.BlockSpec((_BLK, _N_FEAT), lambda i: (i, 0)),
            _small((_N_FEAT, _N_FEAT)),
            _small((_N_FEAT, _N_EXPERTS)),
            _small((_N_FEAT, _N_EXPERTS)),
            _small((_N_FEAT, _N_EXPERTS)),
            _small((_N_FEAT, _N_EXPERTS)),
            _small((_N_SEL, _N_EXPERTS)),
            _small((1, _N_EXPERTS)),
            _small((1, _N_EXPERTS)),
        ],
        out_shape=tuple(out_shape),
        out_specs=tuple(out_specs),
    )


def kernel(rule_features, selected_mask, expert_bias, selected_idx):
    idx_flat = selected_idx.reshape(-1).astype(jnp.int32)
    feat_iota = jnp.arange(_N_FEAT, dtype=jnp.int32)
    onehot = (idx_flat[None, :] == feat_iota[:, None]).astype(jnp.float32)

    slot_iota = jnp.arange(_N_EXPERTS * _N_SEL, dtype=jnp.int32)
    exp_iota = jnp.arange(_N_EXPERTS, dtype=jnp.int32)
    sels = [(slot_iota[:, None] == (_N_SEL * exp_iota + s)[None, :]).astype(jnp.float32)
            for s in range(_N_SEL)]

    mask_t = selected_mask.astype(jnp.float32).T
    count = jnp.maximum(jnp.sum(selected_mask, axis=-1), 1.0)
    cnt2 = count.reshape(1, _N_EXPERTS).astype(jnp.float32)
    bias2 = expert_bias.reshape(1, _N_EXPERTS).astype(jnp.float32)

    x = rule_features.astype(jnp.float32)
    args = (x, onehot, sels[0], sels[1], sels[2], sels[3], mask_t, cnt2, bias2)

    w_erf, l_erf, lo_part, hi_part = _make_call(_THR_ERF, True)(*args)
    lo = jnp.min(lo_part)
    hi = jnp.max(hi_part)
    already = jnp.logical_and(lo >= -1e-06, hi <= 1.0 + 1e-06)

    def _lin_path(operands):
        return _make_call(_THR_LIN, False)(*operands)

    def _erf_path(_):
        return (w_erf, l_erf)

    weights, scaled_logits = jax.lax.cond(already, _lin_path, _erf_path, args)
    return (weights, scaled_logits)
